# Initial kernel scaffold; baseline (speedup 1.0000x reference)
import jax
import jax.numpy as jnp
from jax.experimental import pallas as pl


def kernel(x, edge_index, W1, b1, g1, be1, W2, b2, W3, b3, g2, be2, W4, b4, Wl1, bl1, Wl2, bl2):
    raise NotImplementedError("write your pallas kernel here")


# speedup vs baseline: 11.5587x; 11.5587x over previous
import functools
import math

import jax
import jax.numpy as jnp
from jax import lax
from jax.experimental import pallas as pl
from jax.experimental.pallas import tpu as pltpu
from jax.experimental.pallas import tpu_sc as plsc

F32 = jnp.float32
I32 = jnp.int32
_PREC = lax.Precision.HIGHEST

_BLK_E = 1024
_J = _BLK_E // 128
_ZROWS = 128


def _mk_segsum(N_pad, E_pad, tbl_rows, passes, split_edges, idx_stride,
               out_chunks):
    NC, NS = 2, 16
    per_core = E_pad // NC if split_edges else E_pad
    per_sub = per_core // NS
    assert per_sub % _BLK_E == 0
    n_blocks = per_sub // _BLK_E
    nr_sub = N_pad // NS
    assert N_pad % NS == 0 and nr_sub % _ZROWS == 0
    n_zcopies = nr_sub // _ZROWS

    mesh = plsc.VectorSubcoreMesh(core_axis_name="c", subcore_axis_name="s")

    @functools.partial(
        pl.kernel,
        mesh=mesh,
        compiler_params=pltpu.CompilerParams(use_tc_tiling_on_sc=False),
        out_type=jax.ShapeDtypeStruct((out_chunks, N_pad, 16), F32),
        scratch_types=[
            pltpu.VMEM_SHARED((N_pad, 16), F32),
            pltpu.VMEM((_ZROWS, 16), F32),
            pltpu.VMEM((_BLK_E,), I32),
            pltpu.VMEM((_BLK_E,), I32),
            pltpu.VMEM((_J, 128), I32),
            pltpu.VMEM((_J, 128, 16), F32),
            pltpu.SemaphoreType.DMA,
        ],
    )
    def seg(tbl_hbm, ei_hbm, dst2d_hbm, out_hbm,
            acc, zbuf, srcbuf, gidx, dstbuf, rows, gsem):
        c = lax.axis_index("c")
        s = lax.axis_index("s")
        base_c = c * per_core if split_edges else 0
        r0 = pl.multiple_of(s * nr_sub, _ZROWS)

        def _zb(r, _):
            zbuf[r] = jnp.zeros((16,), F32)
            return 0
        lax.fori_loop(0, _ZROWS, _zb, 0)

        for p in range(passes):
            chunk = c * passes + p
            for i in range(n_zcopies):
                zo = pl.multiple_of(r0 + i * _ZROWS, _ZROWS)
                pltpu.sync_copy(zbuf, acc.at[pl.ds(zo, _ZROWS)])
            plsc.subcore_barrier()

            choff = chunk * idx_stride

            def _block(b, _):
                e0 = pl.multiple_of(base_c + s * per_sub + b * _BLK_E,
                                    _BLK_E)
                row0 = pl.multiple_of(e0 // 128, _J)
                pltpu.sync_copy(ei_hbm.at[0, pl.ds(e0, _BLK_E)], srcbuf)
                pltpu.sync_copy(dst2d_hbm.at[pl.ds(row0, _J)], dstbuf)
                if idx_stride:
                    def _ix(i, _):
                        o = pl.multiple_of(i * 16, 16)
                        v = srcbuf[pl.ds(o, 16)]
                        gidx[pl.ds(o, 16)] = v + choff
                        return 0
                    lax.fori_loop(0, _BLK_E // 16, _ix, 0)
                    isrc = gidx
                else:
                    isrc = srcbuf
                hs = [
                    pltpu.async_copy(
                        tbl_hbm.at[isrc.at[pl.ds(j * 128, 128)]],
                        rows.at[j], gsem)
                    for j in range(_J)
                ]
                for h in hs:
                    h.wait()
                for j in range(_J):
                    pltpu.sync_copy(rows.at[j], acc.at[dstbuf.at[j]],
                                    add=True)
                return 0

            lax.fori_loop(0, n_blocks, _block, 0)
            plsc.subcore_barrier()
            pltpu.sync_copy(acc.at[pl.ds(r0, nr_sub)],
                            out_hbm.at[chunk].at[pl.ds(r0, nr_sub)])

    return seg


def _tc_mlp1(x, acc, a1, c1, W23, N, Bn):
    def body(x_ref, acc_ref, a1_ref, c1_ref, W23_ref, out_ref):
        sa = x_ref[...] + acc_ref[0][:, 0:2] + acc_ref[1][:, 0:2]
        for base, col in ((0, 1), (4, 0)):
            scol = sa[:, col:col + 1]
            u = jax.nn.relu(scol * a1_ref[...] + c1_ref[...])
            Pb = jnp.dot(u, W23_ref[...], preferred_element_type=F32,
                         precision=_PREC)
            for t in range(4):
                out_ref[base + t] = Pb[:, t * 16:(t + 1) * 16]

    grid = (N // Bn,)
    return pl.pallas_call(
        body,
        grid=grid,
        in_specs=[
            pl.BlockSpec((Bn, 2), lambda i: (i, 0)),
            pl.BlockSpec((2, Bn, 16), lambda i: (0, i, 0)),
            pl.BlockSpec((1, 256), lambda i: (0, 0)),
            pl.BlockSpec((1, 256), lambda i: (0, 0)),
            pl.BlockSpec((256, 64), lambda i: (0, 0)),
        ],
        out_specs=pl.BlockSpec((8, Bn, 16), lambda i: (0, i, 0)),
        out_shape=jax.ShapeDtypeStruct((8, N, 16), F32),
    )(x, acc, a1, c1, W23)


def _tc_head(P8, AGG8, acc, bq, b3r, gg, be2r, W4, b4r, Wl1, bl1r, Wl2, bl2r,
             N, Bn):
    def body(P_ref, A_ref, acc_ref, bq_ref, b3_ref, gg_ref, be2_ref,
             W4_ref, b4_ref, Wl1_ref, bl1_ref, Wl2_ref, bl2_ref, out_ref):
        deg = acc_ref[0][:, 2:3] + acc_ref[1][:, 2:3]
        bias = (1.0 + deg) * bq_ref[...] + b3_ref[...]
        halves = []
        for base in (0, 4):
            Pc = jnp.concatenate([P_ref[base + t] for t in range(4)], axis=1)
            Ac = jnp.concatenate([A_ref[base + t] for t in range(4)], axis=1)
            tt = Pc + Ac + bias
            tt = jax.nn.relu(tt * gg_ref[...] + be2_ref[...])
            h2 = jnp.dot(tt, W4_ref[...], preferred_element_type=F32,
                         precision=_PREC) + b4_ref[...]
            halves.append(h2)
        cat = jnp.concatenate(halves, axis=1)
        z = jax.nn.relu(jnp.dot(cat, Wl1_ref[...], preferred_element_type=F32,
                                precision=_PREC) + bl1_ref[...])
        z = jnp.dot(z, Wl2_ref[...], preferred_element_type=F32,
                    precision=_PREC) + bl2_ref[...]
        m = jnp.max(z, axis=1, keepdims=True)
        lse = m + jnp.log(jnp.sum(jnp.exp(z - m), axis=1, keepdims=True))
        out_ref[...] = z - lse

    grid = (N // Bn,)
    w0 = lambda i: (0, 0)
    return pl.pallas_call(
        body,
        grid=grid,
        in_specs=[
            pl.BlockSpec((8, Bn, 16), lambda i: (0, i, 0)),
            pl.BlockSpec((8, Bn, 16), lambda i: (0, i, 0)),
            pl.BlockSpec((2, Bn, 16), lambda i: (0, i, 0)),
            pl.BlockSpec((1, 64), w0),
            pl.BlockSpec((1, 64), w0),
            pl.BlockSpec((1, 64), w0),
            pl.BlockSpec((1, 64), w0),
            pl.BlockSpec((64, 64), w0),
            pl.BlockSpec((1, 64), w0),
            pl.BlockSpec((128, 8), w0),
            pl.BlockSpec((1, 8), w0),
            pl.BlockSpec((8, 8), w0),
            pl.BlockSpec((1, 8), w0),
        ],
        out_specs=pl.BlockSpec((Bn, 8), lambda i: (i, 0)),
        out_shape=jax.ShapeDtypeStruct((N, 8), F32),
    )(P8, AGG8, acc, bq, b3r, gg, be2r, W4, b4r, Wl1, bl1r, Wl2, bl2r)


def kernel(x, edge_index, W1, b1, g1, be1, W2, b2, W3, b3, g2, be2, W4, b4,
           Wl1, bl1, Wl2, bl2):
    N = x.shape[0]
    E = edge_index.shape[1]

    q = 2 * 16 * _BLK_E
    E_pad = ((E + q - 1) // q) * q
    npad = E_pad - E
    if npad:
        pad_src = (jnp.arange(npad, dtype=I32) * 67) % N
        pad_dst = N + (jnp.arange(npad, dtype=I32) % 8)
        ei = jnp.concatenate(
            [edge_index, jnp.stack([pad_src, pad_dst])], axis=1)
    else:
        ei = edge_index
    dst2d = ei[1].reshape(E_pad // 128, 128)

    xa = jnp.concatenate(
        [x, jnp.ones((N, 1), F32), jnp.zeros((N, 13), F32)], axis=1)

    inv = 1.0 / math.sqrt(1.0 + 1e-5)
    a1 = (W1[0] * inv * g1)[None, :]
    c1 = (b1 * inv * g1 + be1)[None, :]
    W23 = jnp.dot(W2, W3, precision=_PREC)
    bq = jnp.dot(b2, W3, precision=_PREC)[None, :]
    b3r = b3[None, :]
    gg = (inv * g2)[None, :]
    be2r = be2[None, :]
    b4r = b4[None, :]
    Wl1p = jnp.zeros((128, 8), F32).at[:, :6].set(Wl1)
    bl1r = jnp.zeros((1, 8), F32).at[:, :6].set(bl1)
    Wl2p = jnp.zeros((8, 8), F32).at[:6, :6].set(Wl2)
    bl2r = jnp.full((1, 8), -1e30, F32).at[:, :6].set(bl2)

    NS_pad = 16 * _ZROWS
    N_pad = ((N + 8 + NS_pad - 1) // NS_pad) * NS_pad

    seg1 = _mk_segsum(N_pad, E_pad, tbl_rows=N, passes=1, split_edges=True,
                      idx_stride=0, out_chunks=2)
    acc2 = seg1(xa, ei, dst2d)

    Bn = 2000
    P8 = _tc_mlp1(x, acc2, a1, c1, W23, N, Bn)

    seg2 = _mk_segsum(N_pad, E_pad, tbl_rows=8 * N, passes=4,
                      split_edges=False, idx_stride=N, out_chunks=8)
    AGG8 = seg2(P8.reshape(8 * N, 16), ei, dst2d)

    out = _tc_head(P8, AGG8, acc2, bq, b3r, gg, be2r, W4, b4r,
                   Wl1p, bl1r, Wl2p, bl2r, N, Bn)
    return out[:, :6]
